# baseline (speedup 1.0000x reference)
import functools

import jax
import jax.numpy as jnp
import numpy as np
from jax import lax
from jax.experimental import pallas as pl
from jax.experimental.pallas import tpu as pltpu
from jax.experimental.pallas import tpu_sc as plsc

_LAYERS = 2
_NC = 2
_NS = 16
_L = 16


def _rel_body(adj_ref, rtab_ref, out_ref):
    adj = adj_ref[...]
    s = jnp.sum(adj, axis=1, keepdims=True)
    acc = jnp.dot(adj, rtab_ref[...], preferred_element_type=jnp.float32)
    out_ref[...] = acc / s


def _rel_aggregate(adj, rtab, bn):
    n, r = adj.shape
    d = rtab.shape[1]
    return pl.pallas_call(
        _rel_body,
        grid=(n // bn,),
        in_specs=[
            pl.BlockSpec((bn, r), lambda i: (i, 0)),
            pl.BlockSpec((r, d), lambda i: (0, 0)),
        ],
        out_specs=pl.BlockSpec((bn, d), lambda i: (i, 0)),
        out_shape=jax.ShapeDtypeStruct((n, d), jnp.float32),
    )(adj, rtab)


def _prop_body(e0_ref, ums_ref, vt_ref, alpha_ref, out_ref):
    e0 = e0_ref[...]
    ums = ums_ref[...]
    vt = vt_ref[...]
    al = alpha_ref[0, 0]
    ep = e0
    for _ in range(_LAYERS):
        t = jnp.dot(vt, ep, preferred_element_type=jnp.float32)
        g = jnp.dot(ums, t, preferred_element_type=jnp.float32)
        ep = jnp.maximum(al * g + (1.0 - al) * ep, 0.0)
    out_ref[...] = ep + e0


def _global_propagate(e0, ums, vt, alpha):
    n, d = e0.shape
    return pl.pallas_call(
        _prop_body,
        out_shape=jax.ShapeDtypeStruct((n, d), jnp.float32),
    )(e0, ums, vt, alpha.reshape(1, 1))


def _gatpre_body(x_ref, w_ref, a1_ref, a2_ref, h_ref, es_ref, ed_ref):
    h = jnp.dot(x_ref[...], w_ref[...], preferred_element_type=jnp.float32)
    h_ref[...] = h
    es_ref[...] = jnp.sum(h * a1_ref[...], axis=1, keepdims=True)
    ed_ref[...] = jnp.sum(h * a2_ref[...], axis=1, keepdims=True)


def _gat_pre(x, w, a1, a2, bn):
    n, d = x.shape
    return pl.pallas_call(
        _gatpre_body,
        grid=(n // bn,),
        in_specs=[
            pl.BlockSpec((bn, d), lambda i: (i, 0)),
            pl.BlockSpec((d, d), lambda i: (0, 0)),
            pl.BlockSpec((1, d), lambda i: (0, 0)),
            pl.BlockSpec((1, d), lambda i: (0, 0)),
        ],
        out_specs=[
            pl.BlockSpec((bn, d), lambda i: (i, 0)),
            pl.BlockSpec((bn, 1), lambda i: (i, 0)),
            pl.BlockSpec((bn, 1), lambda i: (i, 0)),
        ],
        out_shape=[
            jax.ShapeDtypeStruct((n, d), jnp.float32),
            jax.ShapeDtypeStruct((n, 1), jnp.float32),
            jax.ShapeDtypeStruct((n, 1), jnp.float32),
        ],
    )(x, w, a1, a2)


def _gatpost_body(p_ref, sp_ref, out_ref):
    p = jnp.concatenate([p_ref[0], p_ref[1]], axis=1)
    s = sp_ref[...]
    o = p / (s + 1e-16)
    o = jnp.where(o > 0.0, o, jnp.exp(jnp.minimum(o, 0.0)) - 1.0)
    nrm = jnp.sqrt(jnp.sum(o * o, axis=1, keepdims=True))
    nrm = jnp.maximum(nrm, 1e-12)
    out_ref[...] = o / nrm


def _gat_post(p, sp, bn):
    _, n, dh = p.shape
    d = 2 * dh
    return pl.pallas_call(
        _gatpost_body,
        grid=(n // bn,),
        in_specs=[
            pl.BlockSpec((2, bn, dh), lambda i: (0, i, 0)),
            pl.BlockSpec((bn, 1), lambda i: (i, 0)),
        ],
        out_specs=pl.BlockSpec((bn, d), lambda i: (i, 0)),
        out_shape=jax.ShapeDtypeStruct((n, d), jnp.float32),
    )(p, sp.reshape(n, 1))


def _mha_body(x0_ref, x1_ref, x2_ref, rel_ref, wq_ref, wk_ref, wv_ref,
              wo_ref, wp1_ref, wp2_ref, b_ref, out_ref):
    xs = (x0_ref[...], x1_ref[...], x2_ref[...])
    wq = wq_ref[...]
    wk = wk_ref[...]
    wv = wv_ref[...]
    q = [jnp.dot(x, wq, preferred_element_type=jnp.float32) for x in xs]
    k = [jnp.dot(x, wk, preferred_element_type=jnp.float32) for x in xs]
    v = [jnp.dot(x, wv, preferred_element_type=jnp.float32) for x in xs]
    inv_sqrt_d = 1.0 / np.sqrt(np.float32(xs[0].shape[1]))
    logits = [[jnp.sum(q[i] * k[j], axis=1) * inv_sqrt_d for j in range(3)]
              for i in range(3)]
    vbar = jnp.zeros_like(v[0])
    for i in range(3):
        m = jnp.maximum(jnp.maximum(logits[i][0], logits[i][1]), logits[i][2])
        e = [jnp.exp(logits[i][j] - m) for j in range(3)]
        se = e[0] + e[1] + e[2]
        for j in range(3):
            vbar = vbar + (e[j] / se)[:, None] * v[j]
    vbar = vbar * (1.0 / 3.0)
    om = jnp.dot(vbar, wo_ref[...], preferred_element_type=jnp.float32)
    res = (jnp.dot(om, wp1_ref[...], preferred_element_type=jnp.float32)
           + jnp.dot(rel_ref[...], wp2_ref[...], preferred_element_type=jnp.float32)
           + b_ref[...])
    out_ref[...] = jnp.maximum(res, 0.0)


def _mha_final(x0, x1, x2, rel, wq, wk, wv, wo, wp1, wp2, b, bn):
    n, d = x0.shape
    row = lambda i: (i, 0)
    rep = lambda i: (0, 0)
    return pl.pallas_call(
        _mha_body,
        grid=(n // bn,),
        in_specs=[
            pl.BlockSpec((bn, d), row),
            pl.BlockSpec((bn, d), row),
            pl.BlockSpec((bn, d), row),
            pl.BlockSpec((bn, d), row),
            pl.BlockSpec((d, d), rep),
            pl.BlockSpec((d, d), rep),
            pl.BlockSpec((d, d), rep),
            pl.BlockSpec((d, d), rep),
            pl.BlockSpec((d, d), rep),
            pl.BlockSpec((d, d), rep),
            pl.BlockSpec((1, d), rep),
        ],
        out_specs=pl.BlockSpec((bn, d), row),
        out_shape=jax.ShapeDtypeStruct((n, d), jnp.float32),
    )(x0, x1, x2, rel, wq, wk, wv, wo, wp1, wp2, b)


def _gat_edge_sc(src, dst, es, ed, h):
    n, d = h.shape
    e = src.shape[0]
    ept = e // _NS
    c = 80
    nch = ept // c
    nb = 5
    ngrp = nch // nb
    dh = d // _NC
    rpt = (n // _NS) // 8 * 8
    tail = n - _NS * rpt

    src3 = src.reshape(_NS, nch, c)
    dst3 = dst.reshape(_NS, nch, c)
    h2 = jnp.concatenate([h[:, ci * dh:(ci + 1) * dh] for ci in range(_NC)], 0)

    mesh = plsc.VectorSubcoreMesh(
        core_axis_name="c", subcore_axis_name="s",
        num_cores=_NC, num_subcores=_NS)

    nq = dh // _L

    def body(src_h, dst_h, es_h, ed_h, h_h, outp_h, sp_h,
             esv, edv, srcv, dstv, idxb, exb, rowb, zs, out_sh, s_sh, *sems):
        gsem = sems[:nb]
        ssem = sems[nb:2 * nb]
        tsem = sems[2 * nb:]
        cid = lax.axis_index("c")
        sid = lax.axis_index("s")
        zero16 = jnp.zeros((_L,), jnp.float32)
        cbase = cid * n

        pltpu.sync_copy(es_h, esv)
        pltpu.sync_copy(ed_h, edv)
        pltpu.sync_copy(src_h.at[sid], srcv)
        pltpu.sync_copy(dst_h.at[sid], dstv)

        def zrow(r, carry):
            for qq in range(nq):
                rowb[0, r, pl.ds(qq * _L, _L)] = zero16
            return carry
        lax.fori_loop(0, c, zrow, 0)

        def zzs(g, carry):
            zs[pl.ds(g * _L, _L)] = zero16
            return carry
        lax.fori_loop(0, c // _L, zzs, 0)

        base = sid * rpt
        off = 0
        rem = rpt
        while rem > 0:
            sz = min(c, rem)
            pltpu.sync_copy(rowb.at[0, pl.ds(0, sz)],
                            out_sh.at[pl.ds(base + off, sz)])
            off += sz
            rem -= sz

        if tail:
            @pl.when(sid == _NS - 1)
            def _zero_tail():
                pltpu.sync_copy(rowb.at[0, pl.ds(0, tail)],
                                out_sh.at[pl.ds(n - tail, tail)])

        @pl.when(jnp.logical_and(sid == 0, cid == 0))
        def _zero_s():
            for kk in range(n // c):
                pltpu.sync_copy(zs, s_sh.at[pl.ds(kk * c, c)])

        plsc.subcore_barrier()

        def group(g, carry):
            j0 = g * nb
            for b in range(nb):
                j = j0 + b

                @pl.when(g > 0)
                def _drain():
                    pltpu.make_async_copy(
                        rowb.at[b], out_sh.at[dstv.at[j]], ssem[b]).wait()

                    @pl.when(cid == 0)
                    def _drain_s():
                        pltpu.make_async_copy(
                            exb.at[b], s_sh.at[dstv.at[j]], tsem[b]).wait()

                def adj(gg, carry2):
                    sl = pl.ds(gg * _L, _L)
                    idxb[b, sl] = srcv[j, sl] + cbase
                    return carry2
                lax.fori_loop(0, c // _L, adj, 0)
                pltpu.async_copy(h_h.at[idxb.at[b]], rowb.at[b], gsem[b])

            for b in range(nb):
                j = j0 + b

                def cex(gg, carry2):
                    sl = pl.ds(gg * _L, _L)
                    sv = srcv[j, sl]
                    dv = dstv[j, sl]
                    ee = plsc.load_gather(esv, [sv]) + plsc.load_gather(edv, [dv])
                    ee = jnp.where(ee >= 0.0, ee, ee * 0.2)
                    exb[b, sl] = jnp.exp(ee)
                    return carry2
                lax.fori_loop(0, c // _L, cex, 0)

                pltpu.make_async_copy(h_h.at[idxb.at[b]], rowb.at[b],
                                      gsem[b]).wait()

                def sgrp(gg, carry2):
                    wv16 = exb[b, pl.ds(gg * _L, _L)]
                    for lane in range(_L):
                        wvv = wv16.at[jnp.full((_L,), lane, jnp.int32)].get(
                            mode="promise_in_bounds")
                        r = gg * _L + lane
                        for qq in range(nq):
                            sl = pl.ds(qq * _L, _L)
                            rowb[b, r, sl] = rowb[b, r, sl] * wvv
                    return carry2
                lax.fori_loop(0, c // _L, sgrp, 0)

                pltpu.async_copy(rowb.at[b], out_sh.at[dstv.at[j]],
                                 ssem[b], add=True)

                @pl.when(cid == 0)
                def _add_s():
                    pltpu.async_copy(exb.at[b], s_sh.at[dstv.at[j]],
                                     tsem[b], add=True)
            return carry
        lax.fori_loop(0, ngrp, group, 0)

        for b in range(nb):
            jlast = (ngrp - 1) * nb + b
            pltpu.make_async_copy(
                rowb.at[b], out_sh.at[dstv.at[jlast]], ssem[b]).wait()

            @pl.when(cid == 0)
            def _drain_last_s():
                pltpu.make_async_copy(
                    exb.at[b], s_sh.at[dstv.at[jlast]], tsem[b]).wait()

        plsc.subcore_barrier()

        pltpu.sync_copy(out_sh.at[pl.ds(base, rpt)],
                        outp_h.at[cid, pl.ds(base, rpt), :])

        if tail:
            @pl.when(sid == _NS - 1)
            def _copy_tail():
                pltpu.sync_copy(out_sh.at[pl.ds(n - tail, tail)],
                                outp_h.at[cid, pl.ds(n - tail, tail), :])

        @pl.when(jnp.logical_and(sid == 0, cid == 0))
        def _copy_s():
            pltpu.sync_copy(s_sh, sp_h.at[0])

    fn = pl.kernel(
        body,
        out_type=(
            jax.ShapeDtypeStruct((_NC, n, dh), jnp.float32),
            jax.ShapeDtypeStruct((1, n), jnp.float32),
        ),
        mesh=mesh,
        compiler_params=pltpu.CompilerParams(
            needs_layout_passes=False, use_tc_tiling_on_sc=False),
        scratch_types=[
            pltpu.VMEM((n,), jnp.float32),
            pltpu.VMEM((n,), jnp.float32),
            pltpu.VMEM((nch, c), jnp.int32),
            pltpu.VMEM((nch, c), jnp.int32),
            pltpu.VMEM((nb, c), jnp.int32),
            pltpu.VMEM((nb, c), jnp.float32),
            pltpu.VMEM((nb, c, dh), jnp.float32),
            pltpu.VMEM((c,), jnp.float32),
            pltpu.VMEM_SHARED((n, dh), jnp.float32),
            pltpu.VMEM_SHARED((n,), jnp.float32),
        ] + [pltpu.SemaphoreType.DMA] * (3 * nb),
    )
    return fn(src3, dst3, es, ed, h2)


def _gat_layer(x, w, a, src, dst, bn):
    d = x.shape[1]
    a1 = a[:d].reshape(1, d)
    a2 = a[d:].reshape(1, d)
    h, es, ed = _gat_pre(x, w, a1, a2, bn)
    out_p, s_p = _gat_edge_sc(src, dst, es.reshape(-1), ed.reshape(-1), h)
    return _gat_post(out_p, s_p, bn)


def kernel(E_sr, E_tg, R_sr, R_tg, rel_adj_sr, rel_adj_tg, u_mul_s_sr, vt_sr,
           u_mul_s_tg, vt_tg, alpha, W_g1, a_g1, W_g2, a_g2, Wq, Wk, Wv, Wo,
           W_proj, b_proj, edge_index_sr, edge_index_tg):
    n, d = E_sr.shape
    bn = 1000

    rel_sr = _rel_aggregate(rel_adj_sr, R_sr, bn)
    rel_tg = _rel_aggregate(rel_adj_tg, R_tg, bn)

    aug_sr = _global_propagate(E_sr, u_mul_s_sr, vt_sr, alpha)
    aug_tg = _global_propagate(E_tg, u_mul_s_tg, vt_tg, alpha)

    src_s, dst_s = edge_index_sr[0], edge_index_sr[1]
    src_t, dst_t = edge_index_tg[0], edge_index_tg[1]

    xs1 = _gat_layer(E_sr, W_g1, a_g1, src_s, dst_s, bn)
    xt1 = _gat_layer(E_tg, W_g1, a_g1, src_t, dst_t, bn)
    xs2 = _gat_layer(xs1, W_g2, a_g2, src_s, dst_s, bn)
    xt2 = _gat_layer(xt1, W_g2, a_g2, src_t, dst_t, bn)

    wp1 = W_proj[:d]
    wp2 = W_proj[d:]
    b = b_proj.reshape(1, d)

    sr = _mha_final(E_sr, xs1, xs2, rel_sr, Wq, Wk, Wv, Wo, wp1, wp2, b, bn)
    tg = _mha_final(E_tg, xt1, xt2, rel_tg, Wq, Wk, Wv, Wo, wp1, wp2, b, bn)

    return sr, tg, aug_sr, aug_tg


# speedup vs baseline: 1.5282x; 1.5282x over previous
import functools

import jax
import jax.numpy as jnp
import numpy as np
from jax import lax
from jax.experimental import pallas as pl
from jax.experimental.pallas import tpu as pltpu
from jax.experimental.pallas import tpu_sc as plsc

_LAYERS = 2
_NC = 2
_NS = 16
_L = 16


def _rel_body(adj_ref, rtab_ref, out_ref):
    adj = adj_ref[...]
    s = jnp.sum(adj, axis=1, keepdims=True)
    acc = jnp.dot(adj, rtab_ref[...], preferred_element_type=jnp.float32)
    out_ref[...] = acc / s


def _rel_aggregate(adj, rtab, bn):
    n, r = adj.shape
    d = rtab.shape[1]
    return pl.pallas_call(
        _rel_body,
        grid=(n // bn,),
        in_specs=[
            pl.BlockSpec((bn, r), lambda i: (i, 0)),
            pl.BlockSpec((r, d), lambda i: (0, 0)),
        ],
        out_specs=pl.BlockSpec((bn, d), lambda i: (i, 0)),
        out_shape=jax.ShapeDtypeStruct((n, d), jnp.float32),
    )(adj, rtab)


def _prop_body(e0_ref, ums_ref, vt_ref, alpha_ref, out_ref):
    e0 = e0_ref[...]
    ums = ums_ref[...]
    vt = vt_ref[...]
    al = alpha_ref[0, 0]
    ep = e0
    for _ in range(_LAYERS):
        t = jnp.dot(vt, ep, preferred_element_type=jnp.float32)
        g = jnp.dot(ums, t, preferred_element_type=jnp.float32)
        ep = jnp.maximum(al * g + (1.0 - al) * ep, 0.0)
    out_ref[...] = ep + e0


def _global_propagate(e0, ums, vt, alpha):
    n, d = e0.shape
    return pl.pallas_call(
        _prop_body,
        out_shape=jax.ShapeDtypeStruct((n, d), jnp.float32),
    )(e0, ums, vt, alpha.reshape(1, 1))


def _gatpre_body(x_ref, w_ref, a1_ref, a2_ref, h_ref, es_ref, ed_ref):
    h = jnp.dot(x_ref[...], w_ref[...], preferred_element_type=jnp.float32)
    h_ref[...] = h
    es_ref[...] = jnp.sum(h * a1_ref[...], axis=1, keepdims=True)
    ed_ref[...] = jnp.sum(h * a2_ref[...], axis=1, keepdims=True)


def _gat_pre(x, w, a1, a2, bn):
    n, d = x.shape
    return pl.pallas_call(
        _gatpre_body,
        grid=(n // bn,),
        in_specs=[
            pl.BlockSpec((bn, d), lambda i: (i, 0)),
            pl.BlockSpec((d, d), lambda i: (0, 0)),
            pl.BlockSpec((1, d), lambda i: (0, 0)),
            pl.BlockSpec((1, d), lambda i: (0, 0)),
        ],
        out_specs=[
            pl.BlockSpec((bn, d), lambda i: (i, 0)),
            pl.BlockSpec((bn, 1), lambda i: (i, 0)),
            pl.BlockSpec((bn, 1), lambda i: (i, 0)),
        ],
        out_shape=[
            jax.ShapeDtypeStruct((n, d), jnp.float32),
            jax.ShapeDtypeStruct((n, 1), jnp.float32),
            jax.ShapeDtypeStruct((n, 1), jnp.float32),
        ],
    )(x, w, a1, a2)


def _gatpost_body(p_ref, sp_ref, out_ref):
    p = jnp.concatenate([p_ref[0], p_ref[1]], axis=1)
    s = sp_ref[...]
    o = p / (s + 1e-16)
    o = jnp.where(o > 0.0, o, jnp.exp(jnp.minimum(o, 0.0)) - 1.0)
    nrm = jnp.sqrt(jnp.sum(o * o, axis=1, keepdims=True))
    nrm = jnp.maximum(nrm, 1e-12)
    out_ref[...] = o / nrm


def _gat_post(p, sp, bn):
    _, n, dh = p.shape
    d = 2 * dh
    return pl.pallas_call(
        _gatpost_body,
        grid=(n // bn,),
        in_specs=[
            pl.BlockSpec((2, bn, dh), lambda i: (0, i, 0)),
            pl.BlockSpec((bn, 1), lambda i: (i, 0)),
        ],
        out_specs=pl.BlockSpec((bn, d), lambda i: (i, 0)),
        out_shape=jax.ShapeDtypeStruct((n, d), jnp.float32),
    )(p, sp.reshape(n, 1))


def _mha_body(x0_ref, x1_ref, x2_ref, rel_ref, wq_ref, wk_ref, wv_ref,
              wo_ref, wp1_ref, wp2_ref, b_ref, out_ref):
    xs = (x0_ref[...], x1_ref[...], x2_ref[...])
    wq = wq_ref[...]
    wk = wk_ref[...]
    wv = wv_ref[...]
    q = [jnp.dot(x, wq, preferred_element_type=jnp.float32) for x in xs]
    k = [jnp.dot(x, wk, preferred_element_type=jnp.float32) for x in xs]
    v = [jnp.dot(x, wv, preferred_element_type=jnp.float32) for x in xs]
    inv_sqrt_d = 1.0 / np.sqrt(np.float32(xs[0].shape[1]))
    logits = [[jnp.sum(q[i] * k[j], axis=1) * inv_sqrt_d for j in range(3)]
              for i in range(3)]
    vbar = jnp.zeros_like(v[0])
    for i in range(3):
        m = jnp.maximum(jnp.maximum(logits[i][0], logits[i][1]), logits[i][2])
        e = [jnp.exp(logits[i][j] - m) for j in range(3)]
        se = e[0] + e[1] + e[2]
        for j in range(3):
            vbar = vbar + (e[j] / se)[:, None] * v[j]
    vbar = vbar * (1.0 / 3.0)
    om = jnp.dot(vbar, wo_ref[...], preferred_element_type=jnp.float32)
    res = (jnp.dot(om, wp1_ref[...], preferred_element_type=jnp.float32)
           + jnp.dot(rel_ref[...], wp2_ref[...], preferred_element_type=jnp.float32)
           + b_ref[...])
    out_ref[...] = jnp.maximum(res, 0.0)


def _mha_final(x0, x1, x2, rel, wq, wk, wv, wo, wp1, wp2, b, bn):
    n, d = x0.shape
    row = lambda i: (i, 0)
    rep = lambda i: (0, 0)
    return pl.pallas_call(
        _mha_body,
        grid=(n // bn,),
        in_specs=[
            pl.BlockSpec((bn, d), row),
            pl.BlockSpec((bn, d), row),
            pl.BlockSpec((bn, d), row),
            pl.BlockSpec((bn, d), row),
            pl.BlockSpec((d, d), rep),
            pl.BlockSpec((d, d), rep),
            pl.BlockSpec((d, d), rep),
            pl.BlockSpec((d, d), rep),
            pl.BlockSpec((d, d), rep),
            pl.BlockSpec((d, d), rep),
            pl.BlockSpec((1, d), rep),
        ],
        out_specs=pl.BlockSpec((bn, d), row),
        out_shape=jax.ShapeDtypeStruct((n, d), jnp.float32),
    )(x0, x1, x2, rel, wq, wk, wv, wo, wp1, wp2, b)


def _gat_edge_sc(src, dst, es, ed, h):
    n, d = h.shape
    e = src.shape[0]
    ept = e // _NS
    c = 80
    nch = ept // c
    nb = 5
    ngrp = nch // nb
    dh = d // _NC
    rpt = (n // _NS) // 8 * 8
    tail = n - _NS * rpt

    src3 = src.reshape(_NS, nch, c)
    dst3 = dst.reshape(_NS, nch, c)
    h2 = jnp.concatenate([h[:, ci * dh:(ci + 1) * dh] for ci in range(_NC)], 0)

    mesh = plsc.VectorSubcoreMesh(
        core_axis_name="c", subcore_axis_name="s",
        num_cores=_NC, num_subcores=_NS)

    nq = dh // _L

    def body(src_h, dst_h, es_h, ed_h, h_h, outp_h, sp_h,
             esv, edv, srcb, dstb, idxb, exb, rowb, sbuf, zs,
             out_sh, s_sh, *sems):
        gsem = sems[:nb]
        ssem = sems[nb:2 * nb]
        tsem = sems[2 * nb:]
        cid = lax.axis_index("c")
        sid = lax.axis_index("s")
        zero16 = jnp.zeros((_L,), jnp.float32)
        cbase = cid * n

        pltpu.sync_copy(es_h, esv)
        pltpu.sync_copy(ed_h, edv)

        def zrow(r, carry):
            for qq in range(nq):
                sbuf[0, r, pl.ds(qq * _L, _L)] = zero16
            return carry
        lax.fori_loop(0, c, zrow, 0)

        def zzs(g, carry):
            zs[pl.ds(g * _L, _L)] = zero16
            return carry
        lax.fori_loop(0, c // _L, zzs, 0)

        base = sid * rpt
        off = 0
        rem = rpt
        while rem > 0:
            sz = min(c, rem)
            pltpu.sync_copy(sbuf.at[0, pl.ds(0, sz)],
                            out_sh.at[pl.ds(base + off, sz)])
            off += sz
            rem -= sz

        if tail:
            @pl.when(sid == _NS - 1)
            def _zero_tail():
                pltpu.sync_copy(sbuf.at[0, pl.ds(0, tail)],
                                out_sh.at[pl.ds(n - tail, tail)])

        @pl.when(jnp.logical_and(sid == 0, cid == 0))
        def _zero_s():
            for kk in range(n // c):
                pltpu.sync_copy(zs, s_sh.at[pl.ds(kk * c, c)])

        plsc.subcore_barrier()

        def group(g, carry):
            j0 = g * nb
            dpar = (g % 2) * nb

            pltpu.sync_copy(src_h.at[sid, pl.ds(j0, nb), :], srcb)
            pltpu.sync_copy(dst_h.at[sid, pl.ds(j0, nb), :],
                            dstb.at[pl.ds(dpar, nb)])

            for b in range(nb):
                @pl.when(g > 0)
                def _drain():
                    pltpu.make_async_copy(
                        sbuf.at[b], out_sh.at[dstb.at[dpar + b]],
                        ssem[b]).wait()

                    @pl.when(cid == 0)
                    def _drain_s():
                        pltpu.make_async_copy(
                            exb.at[b], s_sh.at[dstb.at[dpar + b]],
                            tsem[b]).wait()

                def adj(gg, carry2):
                    sl = pl.ds(gg * _L, _L)
                    idxb[b, sl] = srcb[b, sl] + cbase
                    return carry2
                lax.fori_loop(0, c // _L, adj, 0)
                pltpu.async_copy(h_h.at[idxb.at[b]], rowb.at[b], gsem[b])

            for b in range(nb):
                def cex(gg, carry2):
                    sl = pl.ds(gg * _L, _L)
                    sv = srcb[b, sl]
                    dv = dstb[dpar + b, sl]
                    ee = plsc.load_gather(esv, [sv]) + plsc.load_gather(edv, [dv])
                    ee = jnp.where(ee >= 0.0, ee, ee * 0.2)
                    exb[b, sl] = jnp.exp(ee)
                    return carry2
                lax.fori_loop(0, c // _L, cex, 0)

                pltpu.make_async_copy(h_h.at[idxb.at[b]], rowb.at[b],
                                      gsem[b]).wait()

                def sgrp(gg, carry2):
                    wv16 = exb[b, pl.ds(gg * _L, _L)]
                    for lane in range(_L):
                        wvv = wv16.at[jnp.full((_L,), lane, jnp.int32)].get(
                            mode="promise_in_bounds")
                        r = gg * _L + lane
                        for qq in range(nq):
                            sl = pl.ds(qq * _L, _L)
                            sbuf[b, r, sl] = rowb[b, r, sl] * wvv
                    return carry2
                lax.fori_loop(0, c // _L, sgrp, 0)

                pltpu.async_copy(sbuf.at[b], out_sh.at[dstb.at[dpar + b]],
                                 ssem[b], add=True)

                @pl.when(cid == 0)
                def _add_s():
                    pltpu.async_copy(exb.at[b], s_sh.at[dstb.at[dpar + b]],
                                     tsem[b], add=True)
            return carry
        lax.fori_loop(0, ngrp, group, 0)

        lpar = ((ngrp - 1) % 2) * nb
        for b in range(nb):
            pltpu.make_async_copy(
                sbuf.at[b], out_sh.at[dstb.at[lpar + b]], ssem[b]).wait()

            @pl.when(cid == 0)
            def _drain_last_s():
                pltpu.make_async_copy(
                    exb.at[b], s_sh.at[dstb.at[lpar + b]], tsem[b]).wait()

        plsc.subcore_barrier()

        pltpu.sync_copy(out_sh.at[pl.ds(base, rpt)],
                        outp_h.at[cid, pl.ds(base, rpt), :])

        if tail:
            @pl.when(sid == _NS - 1)
            def _copy_tail():
                pltpu.sync_copy(out_sh.at[pl.ds(n - tail, tail)],
                                outp_h.at[cid, pl.ds(n - tail, tail), :])

        @pl.when(jnp.logical_and(sid == 0, cid == 0))
        def _copy_s():
            pltpu.sync_copy(s_sh, sp_h.at[0])

    fn = pl.kernel(
        body,
        out_type=(
            jax.ShapeDtypeStruct((_NC, n, dh), jnp.float32),
            jax.ShapeDtypeStruct((1, n), jnp.float32),
        ),
        mesh=mesh,
        compiler_params=pltpu.CompilerParams(
            needs_layout_passes=False, use_tc_tiling_on_sc=False),
        scratch_types=[
            pltpu.VMEM((n,), jnp.float32),
            pltpu.VMEM((n,), jnp.float32),
            pltpu.VMEM((nb, c), jnp.int32),
            pltpu.VMEM((2 * nb, c), jnp.int32),
            pltpu.VMEM((nb, c), jnp.int32),
            pltpu.VMEM((nb, c), jnp.float32),
            pltpu.VMEM((nb, c, dh), jnp.float32),
            pltpu.VMEM((nb, c, dh), jnp.float32),
            pltpu.VMEM((c,), jnp.float32),
            pltpu.VMEM_SHARED((n, dh), jnp.float32),
            pltpu.VMEM_SHARED((n,), jnp.float32),
        ] + [pltpu.SemaphoreType.DMA] * (3 * nb),
    )
    return fn(src3, dst3, es, ed, h2)


def _gat_layer(x, w, a, src, dst, bn):
    d = x.shape[1]
    a1 = a[:d].reshape(1, d)
    a2 = a[d:].reshape(1, d)
    h, es, ed = _gat_pre(x, w, a1, a2, bn)
    out_p, s_p = _gat_edge_sc(src, dst, es.reshape(-1), ed.reshape(-1), h)
    return _gat_post(out_p, s_p, bn)


def kernel(E_sr, E_tg, R_sr, R_tg, rel_adj_sr, rel_adj_tg, u_mul_s_sr, vt_sr,
           u_mul_s_tg, vt_tg, alpha, W_g1, a_g1, W_g2, a_g2, Wq, Wk, Wv, Wo,
           W_proj, b_proj, edge_index_sr, edge_index_tg):
    n, d = E_sr.shape
    bn = 1000

    rel_sr = _rel_aggregate(rel_adj_sr, R_sr, bn)
    rel_tg = _rel_aggregate(rel_adj_tg, R_tg, bn)

    aug_sr = _global_propagate(E_sr, u_mul_s_sr, vt_sr, alpha)
    aug_tg = _global_propagate(E_tg, u_mul_s_tg, vt_tg, alpha)

    src_s, dst_s = edge_index_sr[0], edge_index_sr[1]
    src_t, dst_t = edge_index_tg[0], edge_index_tg[1]

    xs1 = _gat_layer(E_sr, W_g1, a_g1, src_s, dst_s, bn)
    xt1 = _gat_layer(E_tg, W_g1, a_g1, src_t, dst_t, bn)
    xs2 = _gat_layer(xs1, W_g2, a_g2, src_s, dst_s, bn)
    xt2 = _gat_layer(xt1, W_g2, a_g2, src_t, dst_t, bn)

    wp1 = W_proj[:d]
    wp2 = W_proj[d:]
    b = b_proj.reshape(1, d)

    sr = _mha_final(E_sr, xs1, xs2, rel_sr, Wq, Wk, Wv, Wo, wp1, wp2, b, bn)
    tg = _mha_final(E_tg, xt1, xt2, rel_tg, Wq, Wk, Wv, Wo, wp1, wp2, b, bn)

    return sr, tg, aug_sr, aug_tg


# speedup vs baseline: 1.8904x; 1.2370x over previous
import functools

import jax
import jax.numpy as jnp
import numpy as np
from jax import lax
from jax.experimental import pallas as pl
from jax.experimental.pallas import tpu as pltpu
from jax.experimental.pallas import tpu_sc as plsc

_LAYERS = 2
_NC = 2
_NS = 16
_L = 16


def _rel_body(adj_ref, rtab_ref, out_ref):
    adj = adj_ref[...]
    s = jnp.sum(adj, axis=1, keepdims=True)
    acc = jnp.dot(adj, rtab_ref[...], preferred_element_type=jnp.float32)
    out_ref[...] = acc / s


def _rel_aggregate(adj, rtab, bn):
    n, r = adj.shape
    d = rtab.shape[1]
    return pl.pallas_call(
        _rel_body,
        grid=(n // bn,),
        in_specs=[
            pl.BlockSpec((bn, r), lambda i: (i, 0)),
            pl.BlockSpec((r, d), lambda i: (0, 0)),
        ],
        out_specs=pl.BlockSpec((bn, d), lambda i: (i, 0)),
        out_shape=jax.ShapeDtypeStruct((n, d), jnp.float32),
    )(adj, rtab)


def _prop_body(e0_ref, ums_ref, vt_ref, alpha_ref, out_ref):
    e0 = e0_ref[...]
    ums = ums_ref[...]
    vt = vt_ref[...]
    al = alpha_ref[0, 0]
    ep = e0
    for _ in range(_LAYERS):
        t = jnp.dot(vt, ep, preferred_element_type=jnp.float32)
        g = jnp.dot(ums, t, preferred_element_type=jnp.float32)
        ep = jnp.maximum(al * g + (1.0 - al) * ep, 0.0)
    out_ref[...] = ep + e0


def _global_propagate(e0, ums, vt, alpha):
    n, d = e0.shape
    return pl.pallas_call(
        _prop_body,
        out_shape=jax.ShapeDtypeStruct((n, d), jnp.float32),
    )(e0, ums, vt, alpha.reshape(1, 1))


def _gatpre_body(x_ref, w_ref, a1_ref, a2_ref, h_ref, es_ref, ed_ref):
    h = jnp.dot(x_ref[...], w_ref[...], preferred_element_type=jnp.float32)
    h_ref[...] = h
    es_ref[...] = jnp.sum(h * a1_ref[...], axis=1, keepdims=True)
    ed_ref[...] = jnp.sum(h * a2_ref[...], axis=1, keepdims=True)


def _gat_pre(x, w, a1, a2, bn):
    n, d = x.shape
    return pl.pallas_call(
        _gatpre_body,
        grid=(n // bn,),
        in_specs=[
            pl.BlockSpec((bn, d), lambda i: (i, 0)),
            pl.BlockSpec((d, d), lambda i: (0, 0)),
            pl.BlockSpec((1, d), lambda i: (0, 0)),
            pl.BlockSpec((1, d), lambda i: (0, 0)),
        ],
        out_specs=[
            pl.BlockSpec((bn, d), lambda i: (i, 0)),
            pl.BlockSpec((bn, 1), lambda i: (i, 0)),
            pl.BlockSpec((bn, 1), lambda i: (i, 0)),
        ],
        out_shape=[
            jax.ShapeDtypeStruct((n, d), jnp.float32),
            jax.ShapeDtypeStruct((n, 1), jnp.float32),
            jax.ShapeDtypeStruct((n, 1), jnp.float32),
        ],
    )(x, w, a1, a2)


def _gatpost_body(p_ref, sp_ref, out_ref):
    p = jnp.concatenate([p_ref[0], p_ref[1]], axis=1)
    s = jnp.sum(sp_ref[...], axis=1, keepdims=True)
    o = p / (s + 1e-16)
    o = jnp.where(o > 0.0, o, jnp.exp(jnp.minimum(o, 0.0)) - 1.0)
    nrm = jnp.sqrt(jnp.sum(o * o, axis=1, keepdims=True))
    nrm = jnp.maximum(nrm, 1e-12)
    out_ref[...] = o / nrm


def _gat_post(p, sp, bn):
    _, n, dh = p.shape
    d = 2 * dh
    return pl.pallas_call(
        _gatpost_body,
        grid=(n // bn,),
        in_specs=[
            pl.BlockSpec((2, bn, dh), lambda i: (0, i, 0)),
            pl.BlockSpec((bn, 2), lambda i: (i, 0)),
        ],
        out_specs=pl.BlockSpec((bn, d), lambda i: (i, 0)),
        out_shape=jax.ShapeDtypeStruct((n, d), jnp.float32),
    )(p, sp.T)


def _mha_body(x0_ref, x1_ref, x2_ref, rel_ref, wq_ref, wk_ref, wv_ref,
              wo_ref, wp1_ref, wp2_ref, b_ref, out_ref):
    xs = (x0_ref[...], x1_ref[...], x2_ref[...])
    wq = wq_ref[...]
    wk = wk_ref[...]
    wv = wv_ref[...]
    q = [jnp.dot(x, wq, preferred_element_type=jnp.float32) for x in xs]
    k = [jnp.dot(x, wk, preferred_element_type=jnp.float32) for x in xs]
    v = [jnp.dot(x, wv, preferred_element_type=jnp.float32) for x in xs]
    inv_sqrt_d = 1.0 / np.sqrt(np.float32(xs[0].shape[1]))
    logits = [[jnp.sum(q[i] * k[j], axis=1) * inv_sqrt_d for j in range(3)]
              for i in range(3)]
    vbar = jnp.zeros_like(v[0])
    for i in range(3):
        m = jnp.maximum(jnp.maximum(logits[i][0], logits[i][1]), logits[i][2])
        e = [jnp.exp(logits[i][j] - m) for j in range(3)]
        se = e[0] + e[1] + e[2]
        for j in range(3):
            vbar = vbar + (e[j] / se)[:, None] * v[j]
    vbar = vbar * (1.0 / 3.0)
    om = jnp.dot(vbar, wo_ref[...], preferred_element_type=jnp.float32)
    res = (jnp.dot(om, wp1_ref[...], preferred_element_type=jnp.float32)
           + jnp.dot(rel_ref[...], wp2_ref[...], preferred_element_type=jnp.float32)
           + b_ref[...])
    out_ref[...] = jnp.maximum(res, 0.0)


def _mha_final(x0, x1, x2, rel, wq, wk, wv, wo, wp1, wp2, b, bn):
    n, d = x0.shape
    row = lambda i: (i, 0)
    rep = lambda i: (0, 0)
    return pl.pallas_call(
        _mha_body,
        grid=(n // bn,),
        in_specs=[
            pl.BlockSpec((bn, d), row),
            pl.BlockSpec((bn, d), row),
            pl.BlockSpec((bn, d), row),
            pl.BlockSpec((bn, d), row),
            pl.BlockSpec((d, d), rep),
            pl.BlockSpec((d, d), rep),
            pl.BlockSpec((d, d), rep),
            pl.BlockSpec((d, d), rep),
            pl.BlockSpec((d, d), rep),
            pl.BlockSpec((d, d), rep),
            pl.BlockSpec((1, d), rep),
        ],
        out_specs=pl.BlockSpec((bn, d), row),
        out_shape=jax.ShapeDtypeStruct((n, d), jnp.float32),
    )(x0, x1, x2, rel, wq, wk, wv, wo, wp1, wp2, b)


def _gat_edge_sc(src, dst, es, ed, h):
    n, d = h.shape
    e = src.shape[0]
    ept = e // _NS
    c = 80
    nch = ept // c
    nb = 5
    ngrp = nch // nb
    dh = d // _NC
    rpt = (n // _NS) // 8 * 8
    tail = n - _NS * rpt

    src3 = src.reshape(_NS, nch, c)
    dst3 = dst.reshape(_NS, nch, c)
    h2 = jnp.concatenate([h[:, ci * dh:(ci + 1) * dh] for ci in range(_NC)], 0)

    mesh = plsc.VectorSubcoreMesh(
        core_axis_name="c", subcore_axis_name="s",
        num_cores=_NC, num_subcores=_NS)

    nq = dh // _L

    def body(src_h, dst_h, es_h, ed_h, h_h, outp_h, sp_h,
             esv, edv, srcb, dstb, idxb, exb, rowb, sbuf, zs,
             out_sh, s_sh, *sems):
        gsem = sems[:nb]
        ssem = sems[nb:2 * nb]
        tsem = sems[2 * nb:3 * nb]
        cid = lax.axis_index("c")
        sid = lax.axis_index("s")
        zero16 = jnp.zeros((_L,), jnp.float32)
        cbase = cid * n

        pltpu.sync_copy(es_h, esv)
        pltpu.sync_copy(ed_h, edv)

        def zrow(r, carry):
            for qq in range(nq):
                sbuf[0, r, pl.ds(qq * _L, _L)] = zero16
            return carry
        lax.fori_loop(0, c, zrow, 0)

        def zzs(g, carry):
            zs[pl.ds(g * _L, _L)] = zero16
            return carry
        lax.fori_loop(0, c // _L, zzs, 0)

        base = sid * rpt
        off = 0
        rem = rpt
        while rem > 0:
            sz = min(c, rem)
            pltpu.sync_copy(sbuf.at[0, pl.ds(0, sz)],
                            out_sh.at[pl.ds(base + off, sz)])
            off += sz
            rem -= sz

        if tail:
            @pl.when(sid == _NS - 1)
            def _zero_tail():
                pltpu.sync_copy(sbuf.at[0, pl.ds(0, tail)],
                                out_sh.at[pl.ds(n - tail, tail)])

        @pl.when(sid == 0)
        def _zero_s():
            for kk in range(n // c):
                pltpu.sync_copy(zs, s_sh.at[pl.ds(kk * c, c)])

        pltpu.async_copy(src_h.at[sid, pl.ds(0, nb), :],
                         srcb.at[pl.ds(0, nb)], sems[3 * nb])
        pltpu.async_copy(dst_h.at[sid, pl.ds(0, nb), :],
                         dstb.at[pl.ds(0, nb)], sems[3 * nb + 1])

        plsc.subcore_barrier()

        def group(g, carry):
            j0 = g * nb
            dpar = (g % 2) * nb

            pltpu.make_async_copy(src_h.at[sid, pl.ds(j0, nb), :],
                                  srcb.at[pl.ds(dpar, nb)],
                                  sems[3 * nb]).wait()
            pltpu.make_async_copy(dst_h.at[sid, pl.ds(j0, nb), :],
                                  dstb.at[pl.ds(dpar, nb)],
                                  sems[3 * nb + 1]).wait()

            for b in range(nb):
                @pl.when(g > 0)
                def _drain():
                    pltpu.make_async_copy(
                        sbuf.at[b], out_sh.at[dstb.at[dpar + b]],
                        ssem[b]).wait()

                    @pl.when(cid == b % 2)
                    def _drain_s():
                        pltpu.make_async_copy(
                            exb.at[b], s_sh.at[dstb.at[dpar + b]],
                            tsem[b]).wait()

                def adj(gg, carry2):
                    sl = pl.ds(gg * _L, _L)
                    idxb[b, sl] = srcb[dpar + b, sl] + cbase
                    return carry2
                lax.fori_loop(0, c // _L, adj, 0)
                pltpu.async_copy(h_h.at[idxb.at[b]], rowb.at[b], gsem[b])

            @pl.when(g < ngrp - 1)
            def _prefetch():
                npar = nb - dpar
                pltpu.async_copy(src_h.at[sid, pl.ds(j0 + nb, nb), :],
                                 srcb.at[pl.ds(npar, nb)], sems[3 * nb])
                pltpu.async_copy(dst_h.at[sid, pl.ds(j0 + nb, nb), :],
                                 dstb.at[pl.ds(npar, nb)], sems[3 * nb + 1])

            for b in range(nb):
                def cex(gg, carry2):
                    sl = pl.ds(gg * _L, _L)
                    sv = srcb[dpar + b, sl]
                    dv = dstb[dpar + b, sl]
                    ee = plsc.load_gather(esv, [sv]) + plsc.load_gather(edv, [dv])
                    ee = jnp.where(ee >= 0.0, ee, ee * 0.2)
                    exb[b, sl] = jnp.exp(ee)
                    return carry2
                lax.fori_loop(0, c // _L, cex, 0)

                pltpu.make_async_copy(h_h.at[idxb.at[b]], rowb.at[b],
                                      gsem[b]).wait()

                def sgrp(gg, carry2):
                    wv16 = exb[b, pl.ds(gg * _L, _L)]
                    for lane in range(_L):
                        wvv = wv16.at[jnp.full((_L,), lane, jnp.int32)].get(
                            mode="promise_in_bounds")
                        r = gg * _L + lane
                        for qq in range(nq):
                            sl = pl.ds(qq * _L, _L)
                            sbuf[b, r, sl] = rowb[b, r, sl] * wvv
                    return carry2
                lax.fori_loop(0, c // _L, sgrp, 0)

                pltpu.async_copy(sbuf.at[b], out_sh.at[dstb.at[dpar + b]],
                                 ssem[b], add=True)

                @pl.when(cid == b % 2)
                def _add_s():
                    pltpu.async_copy(exb.at[b], s_sh.at[dstb.at[dpar + b]],
                                     tsem[b], add=True)
            return carry
        lax.fori_loop(0, ngrp, group, 0)

        lpar = ((ngrp - 1) % 2) * nb
        for b in range(nb):
            pltpu.make_async_copy(
                sbuf.at[b], out_sh.at[dstb.at[lpar + b]], ssem[b]).wait()

            @pl.when(cid == b % 2)
            def _drain_last_s():
                pltpu.make_async_copy(
                    exb.at[b], s_sh.at[dstb.at[lpar + b]], tsem[b]).wait()

        plsc.subcore_barrier()

        pltpu.sync_copy(out_sh.at[pl.ds(base, rpt)],
                        outp_h.at[cid, pl.ds(base, rpt), :])

        if tail:
            @pl.when(sid == _NS - 1)
            def _copy_tail():
                pltpu.sync_copy(out_sh.at[pl.ds(n - tail, tail)],
                                outp_h.at[cid, pl.ds(n - tail, tail), :])

        @pl.when(sid == 0)
        def _copy_s():
            pltpu.sync_copy(s_sh, sp_h.at[cid])

    fn = pl.kernel(
        body,
        out_type=(
            jax.ShapeDtypeStruct((_NC, n, dh), jnp.float32),
            jax.ShapeDtypeStruct((_NC, n), jnp.float32),
        ),
        mesh=mesh,
        compiler_params=pltpu.CompilerParams(
            needs_layout_passes=False, use_tc_tiling_on_sc=False),
        scratch_types=[
            pltpu.VMEM((n,), jnp.float32),
            pltpu.VMEM((n,), jnp.float32),
            pltpu.VMEM((2 * nb, c), jnp.int32),
            pltpu.VMEM((2 * nb, c), jnp.int32),
            pltpu.VMEM((nb, c), jnp.int32),
            pltpu.VMEM((nb, c), jnp.float32),
            pltpu.VMEM((nb, c, dh), jnp.float32),
            pltpu.VMEM((nb, c, dh), jnp.float32),
            pltpu.VMEM((c,), jnp.float32),
            pltpu.VMEM_SHARED((n, dh), jnp.float32),
            pltpu.VMEM_SHARED((n,), jnp.float32),
        ] + [pltpu.SemaphoreType.DMA] * (3 * nb + 2),
    )
    return fn(src3, dst3, es, ed, h2)


def _gat_layer(x, w, a, src, dst, bn):
    d = x.shape[1]
    a1 = a[:d].reshape(1, d)
    a2 = a[d:].reshape(1, d)
    h, es, ed = _gat_pre(x, w, a1, a2, bn)
    out_p, s_p = _gat_edge_sc(src, dst, es.reshape(-1), ed.reshape(-1), h)
    return _gat_post(out_p, s_p, bn)


def kernel(E_sr, E_tg, R_sr, R_tg, rel_adj_sr, rel_adj_tg, u_mul_s_sr, vt_sr,
           u_mul_s_tg, vt_tg, alpha, W_g1, a_g1, W_g2, a_g2, Wq, Wk, Wv, Wo,
           W_proj, b_proj, edge_index_sr, edge_index_tg):
    n, d = E_sr.shape
    bn = 1000

    rel_sr = _rel_aggregate(rel_adj_sr, R_sr, bn)
    rel_tg = _rel_aggregate(rel_adj_tg, R_tg, bn)

    aug_sr = _global_propagate(E_sr, u_mul_s_sr, vt_sr, alpha)
    aug_tg = _global_propagate(E_tg, u_mul_s_tg, vt_tg, alpha)

    src_s, dst_s = edge_index_sr[0], edge_index_sr[1]
    src_t, dst_t = edge_index_tg[0], edge_index_tg[1]

    xs1 = _gat_layer(E_sr, W_g1, a_g1, src_s, dst_s, bn)
    xt1 = _gat_layer(E_tg, W_g1, a_g1, src_t, dst_t, bn)
    xs2 = _gat_layer(xs1, W_g2, a_g2, src_s, dst_s, bn)
    xt2 = _gat_layer(xt1, W_g2, a_g2, src_t, dst_t, bn)

    wp1 = W_proj[:d]
    wp2 = W_proj[d:]
    b = b_proj.reshape(1, d)

    sr = _mha_final(E_sr, xs1, xs2, rel_sr, Wq, Wk, Wv, Wo, wp1, wp2, b, bn)
    tg = _mha_final(E_tg, xt1, xt2, rel_tg, Wq, Wk, Wv, Wo, wp1, wp2, b, bn)

    return sr, tg, aug_sr, aug_tg
